# baseline (speedup 1.0000x reference)
import jax
import jax.numpy as jnp
from jax.experimental import pallas as pl
from jax.experimental.pallas import tpu as pltpu

NUM_HEADS = 12
SEQ = 256
HEAD_DIM = 64
BPS = 4


def _copy_kernel(revt_ref, out_ref):
    g = pl.program_id(0)
    revt = revt_ref[...]
    base = pltpu.roll(revt, (2 * SEQ - (BPS * g + 1)) % (2 * SEQ), axis=2)
    for db in range(BPS):
        rolled = pltpu.roll(base, 2 * SEQ - db, axis=2) if db else base
        out_ref[:, 1, BPS - 1 - db] = rolled[:, :, 0:SEQ]
        out_ref[:, 0, BPS - 1 - db] = rolled[:, :, SEQ // 2 : 3 * SEQ // 2]


def kernel(seq_length, relative_positional_encoding):
    start = seq_length - SEQ
    sl = jax.lax.dynamic_slice(
        relative_positional_encoding,
        (0, start, 0),
        (NUM_HEADS, 2 * SEQ, HEAD_DIM),
    )
    revt = sl[:, ::-1, :].transpose(0, 2, 1)

    out = pl.pallas_call(
        _copy_kernel,
        grid=(SEQ // 2 // BPS,),
        in_specs=[
            pl.BlockSpec(
                (NUM_HEADS, HEAD_DIM, 2 * SEQ), lambda g: (0, 0, 0)
            ),
        ],
        out_specs=pl.BlockSpec(
            (NUM_HEADS, 2, BPS, HEAD_DIM, SEQ),
            lambda g: (0, 0, (SEQ // 2 // BPS) - 1 - g, 0, 0),
        ),
        out_shape=jax.ShapeDtypeStruct(
            (NUM_HEADS, 2, SEQ // 2, HEAD_DIM, SEQ), jnp.float32
        ),
    )(revt)
    return out.reshape(NUM_HEADS, SEQ, HEAD_DIM, SEQ).transpose(0, 1, 3, 2)


# speedup vs baseline: 1.0889x; 1.0889x over previous
import jax
import jax.numpy as jnp
from jax.experimental import pallas as pl
from jax.experimental.pallas import tpu as pltpu

NUM_HEADS = 12
SEQ = 256
HEAD_DIM = 64
BPS = 8


def _copy_kernel(revt_ref, out_ref):
    g = pl.program_id(0)
    revt = revt_ref[...]
    base = pltpu.roll(revt, (2 * SEQ - (BPS * g + 1)) % (2 * SEQ), axis=2)
    for db in range(BPS):
        win = jax.lax.slice_in_dim(base, db, db + 3 * SEQ // 2, axis=2)
        out_ref[:, 1, BPS - 1 - db] = win[:, :, 0:SEQ]
        out_ref[:, 0, BPS - 1 - db] = win[:, :, SEQ // 2 : 3 * SEQ // 2]


def kernel(seq_length, relative_positional_encoding):
    start = seq_length - SEQ
    sl = jax.lax.dynamic_slice(
        relative_positional_encoding,
        (0, start, 0),
        (NUM_HEADS, 2 * SEQ, HEAD_DIM),
    )
    revt = sl[:, ::-1, :].transpose(0, 2, 1)

    out = pl.pallas_call(
        _copy_kernel,
        grid=(SEQ // 2 // BPS,),
        in_specs=[
            pl.BlockSpec(
                (NUM_HEADS, HEAD_DIM, 2 * SEQ), lambda g: (0, 0, 0)
            ),
        ],
        out_specs=pl.BlockSpec(
            (NUM_HEADS, 2, BPS, HEAD_DIM, SEQ),
            lambda g: (0, 0, (SEQ // 2 // BPS) - 1 - g, 0, 0),
        ),
        out_shape=jax.ShapeDtypeStruct(
            (NUM_HEADS, 2, SEQ // 2, HEAD_DIM, SEQ), jnp.float32
        ),
    )(revt)
    return out.reshape(NUM_HEADS, SEQ, HEAD_DIM, SEQ).transpose(0, 1, 3, 2)
